# baseline (speedup 1.0000x reference)
import functools

import jax
import jax.numpy as jnp
from jax import lax
from jax.experimental import pallas as pl
from jax.experimental.pallas import tpu as pltpu
from jax.experimental.pallas import tpu_sc as plsc

_B, _S, _H = 64, 512, 768
_NL = 4
_K, _D = 1024, 256
_EPS = 1e-12
_N = _B * _S


def _make_sc_gather(n_rows, chunk=128):
    info = plsc.get_sparse_core_info()
    nw = info.num_cores * info.num_subcores
    per_w = n_rows // nw
    n_ch = per_w // chunk
    mesh = plsc.VectorSubcoreMesh(core_axis_name="c", subcore_axis_name="s")

    @functools.partial(
        pl.kernel,
        mesh=mesh,
        out_type=jax.ShapeDtypeStruct((n_rows, _H), jnp.float32),
        scratch_types=[
            pltpu.VMEM((chunk,), jnp.int32),
            pltpu.VMEM((chunk, _H), jnp.float32),
            pltpu.SemaphoreType.DMA,
        ],
    )
    def gather_k(table_hbm, idx_hbm, out_hbm, idx_v, rows_v, sem):
        wid = lax.axis_index("s") * info.num_cores + lax.axis_index("c")
        base = wid * per_w

        def body(c, carry):
            off = pl.multiple_of(base + c * chunk, 8)
            pltpu.sync_copy(idx_hbm.at[pl.ds(off, chunk)], idx_v)
            pltpu.async_copy(table_hbm.at[idx_v], rows_v, sem).wait()
            pltpu.sync_copy(rows_v, out_hbm.at[pl.ds(off, chunk)])
            return carry

        lax.fori_loop(0, n_ch, body, 0)

    return gather_k


def _vq_body(ang_ref, encw_ref, encb_ref, cb_ref, projw_ref, projb_ref,
             g_ref, b_ref, out_ref):
    hi = lax.Precision.HIGHEST
    de = lax.Precision.DEFAULT
    z = lax.dot_general(ang_ref[...], encw_ref[...], (((1,), (0,)), ((), ())),
                        precision=de) + encb_ref[...]
    cb = cb_ref[...]
    zc = lax.dot_general(z, cb, (((1,), (1,)), ((), ())), precision=de)
    ones_row = jnp.ones((1, _D), jnp.float32)
    c2 = lax.dot_general(ones_row, cb * cb, (((1,), (1,)), ((), ())),
                         precision=hi)
    z2 = jnp.sum(z * z, axis=1, keepdims=True)
    d2 = (z2 - 2.0 * zc) + c2
    minv = jnp.min(d2, axis=1, keepdims=True)
    kiota = lax.broadcasted_iota(jnp.int32, d2.shape, 1)
    code = jnp.min(jnp.where(d2 == minv, kiota, _K), axis=1, keepdims=True)
    onehot = (kiota == code).astype(jnp.float32)
    zq = lax.dot_general(onehot, cb, (((1,), (0,)), ((), ())), precision=hi)
    lemb = z + (zq - z)
    h = lax.dot_general(lemb, projw_ref[...], (((1,), (0,)), ((), ())),
                        precision=de) + projb_ref[...]
    mu = jnp.mean(h, axis=1, keepdims=True)
    var = jnp.mean((h - mu) ** 2, axis=1, keepdims=True)
    out_ref[...] = (h - mu) / jnp.sqrt(var + _EPS) * g_ref[...] + b_ref[...]


def _vq_loop_rows(angles2d, enc_W, enc_b, codebook, proj_W, proj_b, g, b):
    return pl.pallas_call(
        _vq_body,
        out_shape=jax.ShapeDtypeStruct((_B * _NL, _H), jnp.float32),
    )(angles2d, enc_W, enc_b, codebook, proj_W, proj_b, g, b)


def _main_body(b0, idx_ref, gath_ref, pos_ref, type0_ref, lnlb_ref, lrows_ref,
               g_ref, b_ref, out_ref):
    ib = b0 + pl.program_id(0)
    x = gath_ref[...] + pos_ref[...] + type0_ref[...]
    comb = jnp.broadcast_to(lnlb_ref[...], (_S, _H))
    siota = lax.broadcasted_iota(jnp.int32, (_S, 1), 0)
    for l in range(_NL):
        p = idx_ref[ib, l]
        row = lrows_ref[0, l]
        comb = jnp.where(siota == p, row[None, :], comb)
    x = x + comb
    mu = jnp.mean(x, axis=1, keepdims=True)
    var = jnp.mean((x - mu) ** 2, axis=1, keepdims=True)
    out_ref[...] = (x - mu) / jnp.sqrt(var + _EPS) * g_ref[...] + b_ref[...]


def _main_part(b0, bp, buf, idx, gath_part, pos, type0, lnlb, lrows, g, b):
    common_in_specs = [
        pl.BlockSpec(memory_space=pltpu.SMEM),
        pl.BlockSpec((_S, _H), lambda i: (i, 0)),
        pl.BlockSpec((_S, _H), lambda i: (0, 0)),
        pl.BlockSpec((1, _H), lambda i: (0, 0)),
        pl.BlockSpec((1, _H), lambda i: (0, 0)),
        pl.BlockSpec((1, _NL, _H), lambda i: (b0 + i, 0, 0)),
        pl.BlockSpec((1, _H), lambda i: (0, 0)),
        pl.BlockSpec((1, _H), lambda i: (0, 0)),
    ]
    out_spec = pl.BlockSpec((_S, _H), lambda i: (b0 + i, 0))
    out_shape = jax.ShapeDtypeStruct((_N, _H), jnp.float32)
    body = functools.partial(_main_body, b0)
    if buf is None:
        return pl.pallas_call(
            body, grid=(bp,), in_specs=common_in_specs, out_specs=out_spec,
            out_shape=out_shape,
        )(idx, gath_part, pos, type0, lnlb, lrows, g, b)

    def body_al(buf_ref, *refs):
        body(*refs)

    return pl.pallas_call(
        body_al, grid=(bp,),
        in_specs=[pl.BlockSpec(memory_space=pl.ANY)] + common_in_specs,
        out_specs=out_spec, out_shape=out_shape,
        input_output_aliases={0: 0},
    )(buf, idx, gath_part, pos, type0, lnlb, lrows, g, b)


def kernel(input_ids, angles, start_loop_indexes, word_emb, pos_emb, type_emb,
           enc_W, enc_b, codebook, proj_W, proj_b,
           ln_loops_g, ln_loops_b, ln_g, ln_b):
    part_batches = [8, 24, 24, 8]
    ids_flat = input_ids.reshape(_N).astype(jnp.int32)
    parts, offs, r0 = [], [], 0
    for bs in part_batches:
        rows = bs * _S
        parts.append(_make_sc_gather(rows)(word_emb, ids_flat[r0:r0 + rows]))
        offs.append(r0 // _S)
        r0 += rows

    lrows = _vq_loop_rows(
        angles.reshape(_B * _NL, -1), enc_W, enc_b.reshape(1, _D), codebook,
        proj_W, proj_b.reshape(1, _H),
        ln_loops_g.reshape(1, _H), ln_loops_b.reshape(1, _H),
    ).reshape(_B, _NL, _H)

    idx = start_loop_indexes.astype(jnp.int32)
    pos = pos_emb[:_S]
    type0 = type_emb[0].reshape(1, _H)
    lnlb = ln_loops_b.reshape(1, _H)
    g = ln_g.reshape(1, _H)
    b = ln_b.reshape(1, _H)

    buf = None
    for p, bs in enumerate(part_batches):
        buf = _main_part(offs[p], bs, buf, idx, parts[p], pos, type0, lnlb,
                         lrows, g, b)
    return buf.reshape(_B, _S, _H)


# speedup vs baseline: 1.0080x; 1.0080x over previous
import functools

import jax
import jax.numpy as jnp
from jax import lax
from jax.experimental import pallas as pl
from jax.experimental.pallas import tpu as pltpu
from jax.experimental.pallas import tpu_sc as plsc

_B, _S, _H = 64, 512, 768
_NL = 4
_K, _D = 1024, 256
_EPS = 1e-12
_N = _B * _S


def _make_sc_gather(n_rows, chunk=128):
    info = plsc.get_sparse_core_info()
    nw = info.num_cores * info.num_subcores
    per_w = n_rows // nw
    n_ch = per_w // chunk
    mesh = plsc.VectorSubcoreMesh(core_axis_name="c", subcore_axis_name="s")

    @functools.partial(
        pl.kernel,
        mesh=mesh,
        out_type=jax.ShapeDtypeStruct((n_rows, _H), jnp.float32),
        scratch_types=[
            pltpu.VMEM((chunk,), jnp.int32),
            pltpu.VMEM((chunk, _H), jnp.float32),
            pltpu.SemaphoreType.DMA,
        ],
    )
    def gather_k(table_hbm, idx_hbm, out_hbm, idx_v, rows_v, sem):
        wid = lax.axis_index("s") * info.num_cores + lax.axis_index("c")
        base = wid * per_w

        def body(c, carry):
            off = pl.multiple_of(base + c * chunk, 8)
            pltpu.sync_copy(idx_hbm.at[pl.ds(off, chunk)], idx_v)
            pltpu.async_copy(table_hbm.at[idx_v], rows_v, sem).wait()
            pltpu.sync_copy(rows_v, out_hbm.at[pl.ds(off, chunk)])
            return carry

        lax.fori_loop(0, n_ch, body, 0)

    return gather_k


def _vq_body(ang_ref, encw_ref, encb_ref, cb_ref, projw_ref, projb_ref,
             g_ref, b_ref, out_ref):
    hi = lax.Precision.HIGHEST
    de = lax.Precision.DEFAULT
    z = lax.dot_general(ang_ref[...], encw_ref[...], (((1,), (0,)), ((), ())),
                        precision=de) + encb_ref[...]
    cb = cb_ref[...]
    zc = lax.dot_general(z, cb, (((1,), (1,)), ((), ())), precision=de)
    ones_row = jnp.ones((1, _D), jnp.float32)
    c2 = lax.dot_general(ones_row, cb * cb, (((1,), (1,)), ((), ())),
                         precision=hi)
    z2 = jnp.sum(z * z, axis=1, keepdims=True)
    d2 = (z2 - 2.0 * zc) + c2
    minv = jnp.min(d2, axis=1, keepdims=True)
    kiota = lax.broadcasted_iota(jnp.int32, d2.shape, 1)
    code = jnp.min(jnp.where(d2 == minv, kiota, _K), axis=1, keepdims=True)
    onehot = (kiota == code).astype(jnp.float32)
    zq = lax.dot_general(onehot, cb, (((1,), (0,)), ((), ())), precision=hi)
    lemb = z + (zq - z)
    h = lax.dot_general(lemb, projw_ref[...], (((1,), (0,)), ((), ())),
                        precision=de) + projb_ref[...]
    mu = jnp.mean(h, axis=1, keepdims=True)
    var = jnp.mean((h - mu) ** 2, axis=1, keepdims=True)
    out_ref[...] = (h - mu) / jnp.sqrt(var + _EPS) * g_ref[...] + b_ref[...]


def _vq_loop_rows(angles2d, enc_W, enc_b, codebook, proj_W, proj_b, g, b):
    return pl.pallas_call(
        _vq_body,
        out_shape=jax.ShapeDtypeStruct((_B * _NL, _H), jnp.float32),
    )(angles2d, enc_W, enc_b, codebook, proj_W, proj_b, g, b)


_BBLK = 2


def _main_body(b0, idx_ref, gath_ref, pos_ref, type0_ref, lnlb_ref, lrows_ref,
               g_ref, b_ref, out_ref):
    ib = b0 + pl.program_id(0) * _BBLK
    rows = _BBLK * _S
    x = (gath_ref[...].reshape(_BBLK, _S, _H) + pos_ref[...][None]
         ).reshape(rows, _H) + type0_ref[...]
    comb = jnp.broadcast_to(lnlb_ref[...], (rows, _H))
    siota = lax.broadcasted_iota(jnp.int32, (rows, 1), 0)
    for bl in range(_BBLK):
        for l in range(_NL):
            p = idx_ref[ib + bl, l] + bl * _S
            row = lrows_ref[bl, l]
            comb = jnp.where(siota == p, row[None, :], comb)
    x = x + comb
    mu = jnp.mean(x, axis=1, keepdims=True)
    var = jnp.mean((x - mu) ** 2, axis=1, keepdims=True)
    out_ref[...] = (x - mu) / jnp.sqrt(var + _EPS) * g_ref[...] + b_ref[...]


def _main_part(b0, bp, buf, idx, gath_part, pos, type0, lnlb, lrows, g, b):
    rows = _BBLK * _S
    common_in_specs = [
        pl.BlockSpec(memory_space=pltpu.SMEM),
        pl.BlockSpec((rows, _H), lambda i: (i, 0)),
        pl.BlockSpec((_S, _H), lambda i: (0, 0)),
        pl.BlockSpec((1, _H), lambda i: (0, 0)),
        pl.BlockSpec((1, _H), lambda i: (0, 0)),
        pl.BlockSpec((_BBLK, _NL, _H), lambda i: (b0 // _BBLK + i, 0, 0)),
        pl.BlockSpec((1, _H), lambda i: (0, 0)),
        pl.BlockSpec((1, _H), lambda i: (0, 0)),
    ]
    out_spec = pl.BlockSpec((rows, _H), lambda i: (b0 // _BBLK + i, 0))
    out_shape = jax.ShapeDtypeStruct((_N, _H), jnp.float32)
    body = functools.partial(_main_body, b0)
    if buf is None:
        return pl.pallas_call(
            body, grid=(bp // _BBLK,), in_specs=common_in_specs,
            out_specs=out_spec,
            out_shape=out_shape,
        )(idx, gath_part, pos, type0, lnlb, lrows, g, b)

    def body_al(buf_ref, *refs):
        body(*refs)

    return pl.pallas_call(
        body_al, grid=(bp // _BBLK,),
        in_specs=[pl.BlockSpec(memory_space=pl.ANY)] + common_in_specs,
        out_specs=out_spec, out_shape=out_shape,
        input_output_aliases={0: 0},
    )(buf, idx, gath_part, pos, type0, lnlb, lrows, g, b)


def kernel(input_ids, angles, start_loop_indexes, word_emb, pos_emb, type_emb,
           enc_W, enc_b, codebook, proj_W, proj_b,
           ln_loops_g, ln_loops_b, ln_g, ln_b):
    part_batches = [8, 24, 24, 8]
    ids_flat = input_ids.reshape(_N).astype(jnp.int32)
    parts, offs, r0 = [], [], 0
    for bs in part_batches:
        rows = bs * _S
        parts.append(_make_sc_gather(rows)(word_emb, ids_flat[r0:r0 + rows]))
        offs.append(r0 // _S)
        r0 += rows

    lrows = _vq_loop_rows(
        angles.reshape(_B * _NL, -1), enc_W, enc_b.reshape(1, _D), codebook,
        proj_W, proj_b.reshape(1, _H),
        ln_loops_g.reshape(1, _H), ln_loops_b.reshape(1, _H),
    ).reshape(_B, _NL, _H)

    idx = start_loop_indexes.astype(jnp.int32)
    pos = pos_emb[:_S]
    type0 = type_emb[0].reshape(1, _H)
    lnlb = ln_loops_b.reshape(1, _H)
    g = ln_g.reshape(1, _H)
    b = ln_b.reshape(1, _H)

    buf = None
    for p, bs in enumerate(part_batches):
        buf = _main_part(offs[p], bs, buf, idx, parts[p], pos, type0, lnlb,
                         lrows, g, b)
    return buf.reshape(_B, _S, _H)
